# Initial kernel scaffold; baseline (speedup 1.0000x reference)
import jax
import jax.numpy as jnp
from jax.experimental import pallas as pl


def kernel(features, labels, similarity_weights):
    raise NotImplementedError("write your pallas kernel here")


# speedup vs baseline: 1.4771x; 1.4771x over previous
import functools

import jax
import jax.numpy as jnp
from jax.experimental import pallas as pl
from jax.experimental.pallas import tpu as pltpu

_TEMP = 0.1
_BASE_TEMP = 0.1
_EPS = 1e-12
_INV_T = 10.0


def _wsc_kernel(fi_ref, fj_ref, sw_ref, lab_ref, out_ref, s_acc, w_acc, p_acc,
                *, bi, bj, nj, cpad):
    i = pl.program_id(0)
    j = pl.program_id(1)

    @pl.when(j == 0)
    def _init():
        s_acc[...] = jnp.zeros_like(s_acc)
        w_acc[...] = jnp.zeros_like(w_acc)
        p_acc[...] = jnp.zeros_like(p_acc)

    fi = fi_ref[...]
    fj = fj_ref[...]
    ri = jax.lax.rsqrt(jnp.maximum(jnp.sum(fi * fi, axis=1, keepdims=True), 1e-24))
    rj = jax.lax.rsqrt(jnp.maximum(jnp.sum(fj * fj, axis=1, keepdims=True), 1e-24))
    fin = fi * (ri * _INV_T)
    fjn = fj * rj
    sim = jax.lax.dot_general(fin, fjn, (((1,), (1,)), ((), ())),
                              preferred_element_type=jnp.float32)

    row_ids = i * bi + jax.lax.broadcasted_iota(jnp.int32, (bi, bj), 0)
    col_ids = j * bj + jax.lax.broadcasted_iota(jnp.int32, (bi, bj), 1)
    offdiag = row_ids != col_ids

    e = jnp.where(offdiag, jnp.exp(sim - _INV_T), 0.0)
    s_acc[...] += jnp.sum(e, axis=1, keepdims=True)

    lab = lab_ref[...]
    oh = (lab == jax.lax.broadcasted_iota(jnp.int32, (cpad, bj), 0)
          ).astype(jnp.float32)
    w = jnp.dot(sw_ref[...], oh, preferred_element_type=jnp.float32)
    w = jnp.where(offdiag, w, 0.0)
    w_acc[...] += jnp.sum(w, axis=1, keepdims=True)
    p_acc[...] += jnp.sum(w * sim, axis=1, keepdims=True)

    @pl.when(j == nj - 1)
    def _emit():
        W = w_acc[...]
        logden = _INV_T + jnp.log(s_acc[...] + _EPS)
        mlpp = (p_acc[...] - W * logden) / (W + _EPS)
        out_ref[...] = -(_TEMP / _BASE_TEMP) * mlpp


@jax.jit
def kernel(features, labels, similarity_weights):
    B, D = features.shape
    C = similarity_weights.shape[1]
    cpad = 128
    bi, bj = 512, 512
    ni, nj = B // bi, B // bj

    lab2d = labels.astype(jnp.int32).reshape(1, B)
    swp = jnp.zeros((B, cpad), jnp.float32).at[:, :C].set(similarity_weights)

    out = pl.pallas_call(
        functools.partial(_wsc_kernel, bi=bi, bj=bj, nj=nj, cpad=cpad),
        grid=(ni, nj),
        in_specs=[
            pl.BlockSpec((bi, D), lambda i, j: (i, 0)),
            pl.BlockSpec((bj, D), lambda i, j: (j, 0)),
            pl.BlockSpec((bi, cpad), lambda i, j: (i, 0)),
            pl.BlockSpec((1, bj), lambda i, j: (0, j)),
        ],
        out_specs=pl.BlockSpec((bi, 1), lambda i, j: (i, 0)),
        out_shape=jax.ShapeDtypeStruct((B, 1), jnp.float32),
        scratch_shapes=[
            pltpu.VMEM((bi, 1), jnp.float32),
            pltpu.VMEM((bi, 1), jnp.float32),
            pltpu.VMEM((bi, 1), jnp.float32),
        ],
        compiler_params=pltpu.CompilerParams(
            dimension_semantics=("parallel", "arbitrary")),
    )(features, features, swp, lab2d)
    return jnp.mean(out)
